# baseline (speedup 1.0000x reference)
import functools

import jax
import jax.numpy as jnp
from jax import lax
from jax.experimental import pallas as pl
from jax.experimental.pallas import tpu as pltpu
from jax.experimental.pallas import tpu_sc as plsc


def _tc_add_body(x_ref, t_ref, o_ref):
    o_ref[...] = x_ref[...] + t_ref[...][None, :, :]


def _tc_kernel(x, table):
    B, S, D = x.shape
    bs = 512
    return pl.pallas_call(
        _tc_add_body,
        grid=(S // bs,),
        in_specs=[
            pl.BlockSpec((B, bs, D), lambda i: (0, i, 0)),
            pl.BlockSpec((bs, D), lambda i: (i, 0)),
        ],
        out_specs=pl.BlockSpec((B, bs, D), lambda i: (0, i, 0)),
        out_shape=jax.ShapeDtypeStruct(x.shape, x.dtype),
    )(x, table)


def _tc_add_body2d(x_ref, t_ref, o_ref):
    o_ref[...] = x_ref[...] + t_ref[...]


def _tc_kernel2d(x, table):
    B, S, D = x.shape
    bs = 512
    x2 = x.reshape(B * S, D)
    nsb = S // bs
    out = pl.pallas_call(
        _tc_add_body2d,
        grid=(nsb, B),
        in_specs=[
            pl.BlockSpec((bs, D), lambda i, b: (b * nsb + i, 0)),
            pl.BlockSpec((bs, D), lambda i, b: (i, 0)),
        ],
        out_specs=pl.BlockSpec((bs, D), lambda i, b: (b * nsb + i, 0)),
        out_shape=jax.ShapeDtypeStruct(x2.shape, x2.dtype),
    )(x2, table)
    return out.reshape(B, S, D)


def _sc_make(N, T, NC, NS, CH):
    NW = NC * NS
    EW = N // NW
    WPB = T // EW
    n_chunks = EW // CH
    mesh = plsc.VectorSubcoreMesh(core_axis_name="c", subcore_axis_name="s")

    @functools.partial(
        pl.kernel,
        out_type=jax.ShapeDtypeStruct((N,), jnp.float32),
        mesh=mesh,
        scratch_types=[
            pltpu.VMEM((CH,), jnp.float32),
            pltpu.VMEM((CH,), jnp.float32),
            pltpu.SemaphoreType.DMA,
            pltpu.SemaphoreType.DMA,
        ],
    )
    def k(x_hbm, t_hbm, o_hbm, xbuf, tbuf, semx, semt):
        wid = lax.axis_index("s") * NC + lax.axis_index("c")
        xbase = wid * EW
        tbase = (wid % WPB) * EW

        def body(i, carry):
            off = i * CH
            cx = pltpu.make_async_copy(
                x_hbm.at[pl.ds(xbase + off, CH)], xbuf, semx)
            ct = pltpu.make_async_copy(
                t_hbm.at[pl.ds(tbase + off, CH)], tbuf, semt)
            cx.start()
            ct.start()
            cx.wait()
            ct.wait()

            def inner(j, c):
                sl = pl.ds(j * 16, 16)
                xbuf[sl] = xbuf[sl] + tbuf[sl]
                return c

            lax.fori_loop(0, CH // 16, inner, 0, unroll=8)
            pltpu.sync_copy(xbuf, o_hbm.at[pl.ds(xbase + off, CH)])
            return carry

        lax.fori_loop(0, n_chunks, body, 0)

    return k


def _sc_kernel(x, table):
    B, S, D = x.shape
    N = B * S * D
    T = S * D
    info = plsc.get_sparse_core_info()
    NC, NS = info.num_cores, info.num_subcores
    out = _sc_make(N, T, NC, NS, 32768)(x.reshape(N), table.reshape(T))
    return out.reshape(B, S, D)


def _sc_make2(B, T, NC, NS, CH):
    NW = NC * NS
    TW = T // NW
    n_tc = TW // CH
    n_it = n_tc * B
    mesh = plsc.VectorSubcoreMesh(core_axis_name="c", subcore_axis_name="s")

    @functools.partial(
        pl.kernel,
        out_type=jax.ShapeDtypeStruct((B * T,), jnp.float32),
        mesh=mesh,
        scratch_types=[
            pltpu.VMEM((2, CH), jnp.float32),
            pltpu.VMEM((2, CH), jnp.float32),
            pltpu.SemaphoreType.DMA,
            pltpu.SemaphoreType.DMA,
            pltpu.SemaphoreType.DMA,
            pltpu.SemaphoreType.DMA,
            pltpu.SemaphoreType.DMA,
            pltpu.SemaphoreType.DMA,
        ],
    )
    def k(x_hbm, t_hbm, o_hbm, xbuf, tbuf, rx0, rx1, rt0, rt1, w0, w1):
        rx = (rx0, rx1)
        rt = (rt0, rt1)
        wsem = (w0, w1)
        wid = lax.axis_index("s") * NC + lax.axis_index("c")
        tbase = wid * TW

        def xoff(i):
            tc, b = i // B, i % B
            return b * T + tbase + tc * CH

        def start_read_x(i):
            pltpu.make_async_copy(
                x_hbm.at[pl.ds(xoff(i), CH)], xbuf.at[i % 2], rx[i % 2]
            ).start()

        def start_read_t(tc):
            pltpu.make_async_copy(
                t_hbm.at[pl.ds(tbase + tc * CH, CH)], tbuf.at[tc % 2], rt[tc % 2]
            ).start()

        start_read_x(0)
        start_read_t(0)

        for i in range(n_it):
            c = i % 2
            tc, b = i // B, i % B
            if i + 1 < n_it:
                if i >= 1:
                    pltpu.make_async_copy(
                        xbuf.at[(i + 1) % 2],
                        o_hbm.at[pl.ds(xoff(i - 1), CH)],
                        wsem[(i + 1) % 2],
                    ).wait()
                start_read_x(i + 1)
                if (i + 1) // B != tc:
                    start_read_t((i + 1) // B)
            pltpu.make_async_copy(
                x_hbm.at[pl.ds(xoff(i), CH)], xbuf.at[c], rx[c]
            ).wait()
            if b == 0:
                pltpu.make_async_copy(
                    t_hbm.at[pl.ds(tbase + tc * CH, CH)],
                    tbuf.at[tc % 2],
                    rt[tc % 2],
                ).wait()

            xb = xbuf.at[c]
            tb = tbuf.at[tc % 2]

            def inner(j, carry):
                sl = pl.ds(j * 16, 16)
                xb[sl] = xb[sl] + tb[sl]
                return carry

            lax.fori_loop(0, CH // 16, inner, 0, unroll=8)

            pltpu.make_async_copy(
                xbuf.at[c], o_hbm.at[pl.ds(xoff(i), CH)], wsem[c]
            ).start()

        pltpu.make_async_copy(
            xbuf.at[(n_it - 2) % 2],
            o_hbm.at[pl.ds(xoff(n_it - 2), CH)],
            wsem[(n_it - 2) % 2],
        ).wait()
        pltpu.make_async_copy(
            xbuf.at[(n_it - 1) % 2],
            o_hbm.at[pl.ds(xoff(n_it - 1), CH)],
            wsem[(n_it - 1) % 2],
        ).wait()

    return k


def _sc_kernel2(x, table, CH=16384):
    B, S, D = x.shape
    T = S * D
    info = plsc.get_sparse_core_info()
    NC, NS = info.num_cores, info.num_subcores
    out = _sc_make2(B, T, NC, NS, CH)(x.reshape(B * T), table.reshape(T))
    return out.reshape(B, S, D)


def _tc_partial(x, table, nb):
    B, S, D = x.shape
    bs = 512
    return pl.pallas_call(
        _tc_add_body,
        grid=(S // bs,),
        in_specs=[
            pl.BlockSpec((nb, bs, D), lambda i: (0, i, 0)),
            pl.BlockSpec((bs, D), lambda i: (i, 0)),
        ],
        out_specs=pl.BlockSpec((nb, bs, D), lambda i: (0, i, 0)),
        out_shape=jax.ShapeDtypeStruct((nb, S, D), x.dtype),
    )(x[:nb], table)


def _concurrency_diag(x, table):
    B, S, D = x.shape
    T = S * D
    info = plsc.get_sparse_core_info()
    NC, NS = info.num_cores, info.num_subcores
    sc_out = _sc_make(T, T, NC, NS, 32768)(x[B - 1].reshape(T), table.reshape(T))
    tc_out = _tc_partial(x, table, B - 1)
    return tc_out, sc_out


def kernel(x, table):
    return _sc_kernel2(x, table)


# speedup vs baseline: 1.4611x; 1.4611x over previous
import functools

import jax
import jax.numpy as jnp
from jax import lax
from jax.experimental import pallas as pl
from jax.experimental.pallas import tpu as pltpu
from jax.experimental.pallas import tpu_sc as plsc


def _tc_add_body(x_ref, t_ref, o_ref):
    o_ref[...] = x_ref[...] + t_ref[...][None, :, :]


def _tc_kernel(x, table):
    B, S, D = x.shape
    bs = 512
    return pl.pallas_call(
        _tc_add_body,
        grid=(S // bs,),
        in_specs=[
            pl.BlockSpec((B, bs, D), lambda i: (0, i, 0)),
            pl.BlockSpec((bs, D), lambda i: (i, 0)),
        ],
        out_specs=pl.BlockSpec((B, bs, D), lambda i: (0, i, 0)),
        out_shape=jax.ShapeDtypeStruct(x.shape, x.dtype),
    )(x, table)


def _tc_add_body2d(x_ref, t_ref, o_ref):
    o_ref[...] = x_ref[...] + t_ref[...]


def _tc_kernel2d(x, table):
    B, S, D = x.shape
    bs = 512
    x2 = x.reshape(B * S, D)
    nsb = S // bs
    out = pl.pallas_call(
        _tc_add_body2d,
        grid=(nsb, B),
        in_specs=[
            pl.BlockSpec((bs, D), lambda i, b: (b * nsb + i, 0)),
            pl.BlockSpec((bs, D), lambda i, b: (i, 0)),
        ],
        out_specs=pl.BlockSpec((bs, D), lambda i, b: (b * nsb + i, 0)),
        out_shape=jax.ShapeDtypeStruct(x2.shape, x2.dtype),
    )(x2, table)
    return out.reshape(B, S, D)


def _sc_make(N, T, NC, NS, CH):
    NW = NC * NS
    EW = N // NW
    WPB = T // EW
    n_chunks = EW // CH
    mesh = plsc.VectorSubcoreMesh(core_axis_name="c", subcore_axis_name="s")

    @functools.partial(
        pl.kernel,
        out_type=jax.ShapeDtypeStruct((N,), jnp.float32),
        mesh=mesh,
        scratch_types=[
            pltpu.VMEM((CH,), jnp.float32),
            pltpu.VMEM((CH,), jnp.float32),
            pltpu.SemaphoreType.DMA,
            pltpu.SemaphoreType.DMA,
        ],
    )
    def k(x_hbm, t_hbm, o_hbm, xbuf, tbuf, semx, semt):
        wid = lax.axis_index("s") * NC + lax.axis_index("c")
        xbase = wid * EW
        tbase = (wid % WPB) * EW

        def body(i, carry):
            off = i * CH
            cx = pltpu.make_async_copy(
                x_hbm.at[pl.ds(xbase + off, CH)], xbuf, semx)
            ct = pltpu.make_async_copy(
                t_hbm.at[pl.ds(tbase + off, CH)], tbuf, semt)
            cx.start()
            ct.start()
            cx.wait()
            ct.wait()

            def inner(j, c):
                sl = pl.ds(j * 16, 16)
                xbuf[sl] = xbuf[sl] + tbuf[sl]
                return c

            lax.fori_loop(0, CH // 16, inner, 0, unroll=8)
            pltpu.sync_copy(xbuf, o_hbm.at[pl.ds(xbase + off, CH)])
            return carry

        lax.fori_loop(0, n_chunks, body, 0)

    return k


def _sc_kernel(x, table):
    B, S, D = x.shape
    N = B * S * D
    T = S * D
    info = plsc.get_sparse_core_info()
    NC, NS = info.num_cores, info.num_subcores
    out = _sc_make(N, T, NC, NS, 32768)(x.reshape(N), table.reshape(T))
    return out.reshape(B, S, D)


def _sc_make2(B, T, NC, NS, CH):
    NW = NC * NS
    TW = T // NW
    n_tc = TW // CH
    n_it = n_tc * B
    mesh = plsc.VectorSubcoreMesh(core_axis_name="c", subcore_axis_name="s")

    @functools.partial(
        pl.kernel,
        out_type=jax.ShapeDtypeStruct((B * T,), jnp.float32),
        mesh=mesh,
        scratch_types=[
            pltpu.VMEM((2, CH), jnp.float32),
            pltpu.VMEM((2, CH), jnp.float32),
            pltpu.SemaphoreType.DMA,
            pltpu.SemaphoreType.DMA,
            pltpu.SemaphoreType.DMA,
            pltpu.SemaphoreType.DMA,
            pltpu.SemaphoreType.DMA,
            pltpu.SemaphoreType.DMA,
        ],
    )
    def k(x_hbm, t_hbm, o_hbm, xbuf, tbuf, rx0, rx1, rt0, rt1, w0, w1):
        rx = (rx0, rx1)
        rt = (rt0, rt1)
        wsem = (w0, w1)
        wid = lax.axis_index("s") * NC + lax.axis_index("c")
        tbase = wid * TW

        def xoff(i):
            tc, b = i // B, i % B
            return b * T + tbase + tc * CH

        def start_read_x(i):
            pltpu.make_async_copy(
                x_hbm.at[pl.ds(xoff(i), CH)], xbuf.at[i % 2], rx[i % 2]
            ).start()

        def start_read_t(tc):
            pltpu.make_async_copy(
                t_hbm.at[pl.ds(tbase + tc * CH, CH)], tbuf.at[tc % 2], rt[tc % 2]
            ).start()

        start_read_x(0)
        start_read_t(0)

        for i in range(n_it):
            c = i % 2
            tc, b = i // B, i % B
            if i + 1 < n_it:
                if i >= 1:
                    pltpu.make_async_copy(
                        xbuf.at[(i + 1) % 2],
                        o_hbm.at[pl.ds(xoff(i - 1), CH)],
                        wsem[(i + 1) % 2],
                    ).wait()
                start_read_x(i + 1)
                if (i + 1) // B != tc:
                    start_read_t((i + 1) // B)
            pltpu.make_async_copy(
                x_hbm.at[pl.ds(xoff(i), CH)], xbuf.at[c], rx[c]
            ).wait()
            if b == 0:
                pltpu.make_async_copy(
                    t_hbm.at[pl.ds(tbase + tc * CH, CH)],
                    tbuf.at[tc % 2],
                    rt[tc % 2],
                ).wait()

            xb = xbuf.at[c]
            tb = tbuf.at[tc % 2]

            @plsc.parallel_loop(0, CH // 16, 1, unroll=8)
            def inner(j):
                sl = pl.ds(j * 16, 16)
                xb[sl] = xb[sl] + tb[sl]

            pltpu.make_async_copy(
                xbuf.at[c], o_hbm.at[pl.ds(xoff(i), CH)], wsem[c]
            ).start()

        pltpu.make_async_copy(
            xbuf.at[(n_it - 2) % 2],
            o_hbm.at[pl.ds(xoff(n_it - 2), CH)],
            wsem[(n_it - 2) % 2],
        ).wait()
        pltpu.make_async_copy(
            xbuf.at[(n_it - 1) % 2],
            o_hbm.at[pl.ds(xoff(n_it - 1), CH)],
            wsem[(n_it - 1) % 2],
        ).wait()

    return k


def _sc_kernel2(x, table, CH=16384):
    B, S, D = x.shape
    T = S * D
    info = plsc.get_sparse_core_info()
    NC, NS = info.num_cores, info.num_subcores
    out = _sc_make2(B, T, NC, NS, CH)(x.reshape(B * T), table.reshape(T))
    return out.reshape(B, S, D)


def _tc_partial(x, table, nb):
    B, S, D = x.shape
    bs = 512
    return pl.pallas_call(
        _tc_add_body,
        grid=(S // bs,),
        in_specs=[
            pl.BlockSpec((nb, bs, D), lambda i: (0, i, 0)),
            pl.BlockSpec((bs, D), lambda i: (i, 0)),
        ],
        out_specs=pl.BlockSpec((nb, bs, D), lambda i: (0, i, 0)),
        out_shape=jax.ShapeDtypeStruct((nb, S, D), x.dtype),
    )(x[:nb], table)


def _concurrency_diag(x, table):
    B, S, D = x.shape
    T = S * D
    info = plsc.get_sparse_core_info()
    NC, NS = info.num_cores, info.num_subcores
    sc_out = _sc_make(T, T, NC, NS, 32768)(x[B - 1].reshape(T), table.reshape(T))
    tc_out = _tc_partial(x, table, B - 1)
    return tc_out, sc_out


def kernel(x, table):
    return _sc_kernel2(x, table)


# speedup vs baseline: 7.5314x; 5.1545x over previous
import jax
import jax.numpy as jnp
from jax.experimental import pallas as pl


def _add_body(x_ref, t_ref, o_ref):
    o_ref[...] = x_ref[...] + t_ref[...][None, :, :]


def kernel(x, table):
    B, S, D = x.shape
    bs = 512
    return pl.pallas_call(
        _add_body,
        grid=(S // bs,),
        in_specs=[
            pl.BlockSpec((B, bs, D), lambda i: (0, i, 0)),
            pl.BlockSpec((bs, D), lambda i: (i, 0)),
        ],
        out_specs=pl.BlockSpec((B, bs, D), lambda i: (0, i, 0)),
        out_shape=jax.ShapeDtypeStruct(x.shape, x.dtype),
    )(x, table)
